# baseline (speedup 1.0000x reference)
import functools

import jax
import jax.numpy as jnp
from jax import lax
from jax.experimental import pallas as pl
from jax.experimental.pallas import tpu as pltpu
from jax.experimental.pallas import tpu_sc as plsc

B = 16384
F = 64
MU = 3.5
NC = 2
NS = 16
NW = NC * NS
BPW = B // NW
CHUNK = 128
NCHUNK = BPW // CHUNK
L = 16


def _sc_body(uidx_hbm, iidx_hbm, eu_hbm, ei_hbm, bu_hbm, bi_hbm, out_hbm,
             uidx_v, iidx_v, eurows_v, eirows_v, bu_v, bi_v, out_v, sem):
    wid = lax.axis_index("s") * NC + lax.axis_index("c")
    base = wid * BPW

    copies = []
    for c in range(NCHUNK):
        off = base + c * CHUNK
        pltpu.sync_copy(uidx_hbm.at[pl.ds(off, CHUNK)], uidx_v.at[c])
        pltpu.sync_copy(iidx_hbm.at[pl.ds(off, CHUNK)], iidx_v.at[c])
        copies.append(pltpu.async_copy(eu_hbm.at[uidx_v.at[c]], eurows_v.at[c], sem))
        copies.append(pltpu.async_copy(ei_hbm.at[iidx_v.at[c]], eirows_v.at[c], sem))
        copies.append(pltpu.async_copy(bu_hbm.at[uidx_v.at[c]], bu_v.at[c], sem))
        copies.append(pltpu.async_copy(bi_hbm.at[iidx_v.at[c]], bi_v.at[c], sem))
    for cp in copies:
        cp.wait()

    lane = lax.iota(jnp.int32, L)
    for c in range(NCHUNK):
        def grp_body(g, _, c=c):
            out_vec = jnp.zeros((L,), jnp.float32)
            for i in range(L):
                r = g * L + i
                acc = (eurows_v[c, r, pl.ds(0, L)] * eirows_v[c, r, pl.ds(0, L)])
                for k in range(1, F // L):
                    acc = acc + (eurows_v[c, r, pl.ds(k * L, L)]
                                 * eirows_v[c, r, pl.ds(k * L, L)])
                s = jnp.sum(acc)
                out_vec = jnp.where(lane == i, s, out_vec)
            tot = (out_vec + bu_v[c, pl.ds(g * L, L)] + bi_v[c, pl.ds(g * L, L)]
                   + jnp.float32(MU))
            out_v[c, pl.ds(g * L, L)] = tot
            return 0
        lax.fori_loop(0, CHUNK // L, grp_body, 0)
        pltpu.sync_copy(out_v.at[c], out_hbm.at[pl.ds(base + c * CHUNK, CHUNK)])


@functools.partial(
    pl.kernel,
    mesh=plsc.VectorSubcoreMesh(core_axis_name="c", subcore_axis_name="s"),
    out_type=jax.ShapeDtypeStruct((B,), jnp.float32),
    compiler_params=pltpu.CompilerParams(
        needs_layout_passes=False, use_tc_tiling_on_sc=False),
    scratch_types=[
        pltpu.VMEM((NCHUNK, CHUNK), jnp.int32),
        pltpu.VMEM((NCHUNK, CHUNK), jnp.int32),
        pltpu.VMEM((NCHUNK, CHUNK, F), jnp.float32),
        pltpu.VMEM((NCHUNK, CHUNK, F), jnp.float32),
        pltpu.VMEM((NCHUNK, CHUNK), jnp.float32),
        pltpu.VMEM((NCHUNK, CHUNK), jnp.float32),
        pltpu.VMEM((NCHUNK, CHUNK), jnp.float32),
        pltpu.SemaphoreType.DMA,
    ],
)
def _mf_score(uidx_hbm, iidx_hbm, eu_hbm, ei_hbm, bu_hbm, bi_hbm, out_hbm,
              uidx_v, iidx_v, eurows_v, eirows_v, bu_v, bi_v, out_v, sem):
    _sc_body(uidx_hbm, iidx_hbm, eu_hbm, ei_hbm, bu_hbm, bi_hbm, out_hbm,
             uidx_v, iidx_v, eurows_v, eirows_v, bu_v, bi_v, out_v, sem)


def kernel(user_idx, item_idx, embed_user, embed_item, user_bias, item_bias):
    return _mf_score(user_idx, item_idx, embed_user, embed_item,
                     user_bias[:, 0], item_bias[:, 0])


# speedup vs baseline: 2.2033x; 2.2033x over previous
import functools

import jax
import jax.numpy as jnp
from jax import lax
from jax.experimental import pallas as pl
from jax.experimental.pallas import tpu as pltpu
from jax.experimental.pallas import tpu_sc as plsc

B = 16384
F = 64
MU = 3.5
NC = 2
NS = 16
NW = NC * NS
BPW = B // NW
CHUNK = 128
NCHUNK = BPW // CHUNK
L = 16
NSLOT = 4


def _scalar_at(vec_ref, e):
    lane = lax.iota(jnp.int32, L)
    g = pl.multiple_of((e // L) * L, L)
    v = vec_ref[pl.ds(g, L)]
    return jnp.sum(jnp.where(lane == (e % L), v, 0))


def _issue_pair(euT, eiT, uslot, islot, sem, uidx_v, iidx_v, e):
    u = _scalar_at(uidx_v, e)
    i = _scalar_at(iidx_v, e)
    ju = pl.multiple_of((u // 128) * 128, 128)
    ji = pl.multiple_of((i // 128) * 128, 128)
    pltpu.async_copy(euT.at[:, pl.ds(ju, 128)], uslot, sem)
    pltpu.async_copy(eiT.at[:, pl.ds(ji, 128)], islot, sem)


def _sc_body(uidx_hbm, iidx_hbm, euT, eiT, bu_hbm, bi_hbm, out_hbm,
             uidx_v, iidx_v, uslots, islots,
             bu_v, bi_v, out_v, sems):
    wid = lax.axis_index("s") * NC + lax.axis_index("c")
    base = wid * BPW

    pltpu.sync_copy(uidx_hbm.at[pl.ds(base, BPW)], uidx_v)
    pltpu.sync_copy(iidx_hbm.at[pl.ds(base, BPW)], iidx_v)

    bias_copies = []
    for c in range(NCHUNK):
        sl = pl.ds(c * CHUNK, CHUNK)
        bias_copies.append(
            pltpu.async_copy(bu_hbm.at[uidx_v.at[sl]], bu_v.at[sl], sems[NSLOT]))
        bias_copies.append(
            pltpu.async_copy(bi_hbm.at[iidx_v.at[sl]], bi_v.at[sl], sems[NSLOT]))
    for cp in bias_copies:
        cp.wait()

    for s in range(NSLOT):
        _issue_pair(euT, eiT, uslots.at[s], islots.at[s], sems[s],
                    uidx_v, iidx_v, s)

    lane = lax.iota(jnp.int32, L)

    def make_branch(s):
        def branch(r, out_vec):
            uslot = uslots.at[s]
            islot = islots.at[s]
            pltpu.make_async_copy(euT.at[:, pl.ds(0, 128)], uslot, sems[s]).wait()
            pltpu.make_async_copy(eiT.at[:, pl.ds(0, 128)], islot, sems[s]).wait()
            u = _scalar_at(uidx_v, r)
            i = _scalar_at(iidx_v, r)
            ul = jnp.full((L,), u % 128, jnp.int32)
            il = jnp.full((L,), i % 128, jnp.int32)
            acc = jnp.zeros((L,), jnp.float32)
            for a in range(F // L):
                rows = lane + (a * L)
                vu = plsc.load_gather(uslot, [rows, ul])
                vi = plsc.load_gather(islot, [rows, il])
                acc = acc + vu * vi
            sval = jnp.sum(acc)
            out_vec = jnp.where(lane == (r % L), sval, out_vec)
            e = r + NSLOT

            @pl.when(e < BPW)
            def _():
                _issue_pair(euT, eiT, uslot, islot, sems[s], uidx_v, iidx_v, e)
            return out_vec
        return branch

    branches = [make_branch(s) for s in range(NSLOT)]

    def body(r, out_vec):
        out_vec = lax.switch(r % NSLOT, branches, r, out_vec)

        @pl.when(r % L == L - 1)
        def _():
            g16 = pl.multiple_of((r // L) * L, L)
            sl = pl.ds(g16, L)
            out_v[sl] = out_vec + bu_v[sl] + bi_v[sl] + jnp.float32(MU)
        return jnp.where(r % L == L - 1, jnp.zeros((L,), jnp.float32), out_vec)

    lax.fori_loop(0, BPW, body, jnp.zeros((L,), jnp.float32))
    pltpu.sync_copy(out_v, out_hbm.at[pl.ds(base, BPW)])


@functools.partial(
    pl.kernel,
    mesh=plsc.VectorSubcoreMesh(core_axis_name="c", subcore_axis_name="s"),
    out_type=jax.ShapeDtypeStruct((B,), jnp.float32),
    compiler_params=pltpu.CompilerParams(
        needs_layout_passes=False, use_tc_tiling_on_sc=True),
    scratch_types=[
        pltpu.VMEM((BPW,), jnp.int32),
        pltpu.VMEM((BPW,), jnp.int32),
        pltpu.VMEM((NSLOT, F, 128), jnp.float32),
        pltpu.VMEM((NSLOT, F, 128), jnp.float32),
        pltpu.VMEM((BPW,), jnp.float32),
        pltpu.VMEM((BPW,), jnp.float32),
        pltpu.VMEM((BPW,), jnp.float32),
        [pltpu.SemaphoreType.DMA] * (NSLOT + 1),
    ],
)
def _mf3(uidx_hbm, iidx_hbm, euT, eiT, bu_hbm, bi_hbm, out_hbm,
         uidx_v, iidx_v, uslots, islots, bu_v, bi_v, out_v, sems):
    _sc_body(uidx_hbm, iidx_hbm, euT, eiT, bu_hbm, bi_hbm, out_hbm,
             uidx_v, iidx_v, uslots, islots, bu_v, bi_v, out_v, sems)


def kernel(user_idx, item_idx, embed_user, embed_item, user_bias, item_bias):
    return _mf3(user_idx, item_idx, embed_user.T, embed_item.T,
                user_bias[:, 0], item_bias[:, 0])


# speedup vs baseline: 2.8710x; 1.3030x over previous
import functools

import jax
import jax.numpy as jnp
from jax import lax
from jax.experimental import pallas as pl
from jax.experimental.pallas import tpu as pltpu
from jax.experimental.pallas import tpu_sc as plsc

B = 16384
F = 64
MU = 3.5
NC = 2
NS = 16
NW = NC * NS
BPW = B // NW
CHUNK = 128
NCHUNK = BPW // CHUNK
L = 16
NSLOT = 6


def _scalar_at(vec_ref, e):
    lane = lax.iota(jnp.int32, L)
    g = pl.multiple_of((e // L) * L, L)
    v = vec_ref[pl.ds(g, L)]
    return jnp.sum(jnp.where(lane == (e % L), v, 0))


def _issue_pair(euT, eiT, uslot, islot, sem, uidx_v, iidx_v, e):
    u = _scalar_at(uidx_v, e)
    i = _scalar_at(iidx_v, e)
    ju = pl.multiple_of((u // 128) * 128, 128)
    ji = pl.multiple_of((i // 128) * 128, 128)
    pltpu.async_copy(euT.at[:, pl.ds(ju, 128)], uslot, sem)
    pltpu.async_copy(eiT.at[:, pl.ds(ji, 128)], islot, sem)


def _dots_body(uidx_hbm, iidx_hbm, euT, eiT, out_hbm,
               uidx_v, iidx_v, uslots, islots, out_v, sems):
    wid = lax.axis_index("s") * NC + lax.axis_index("c")
    base = wid * BPW

    pltpu.sync_copy(uidx_hbm.at[pl.ds(base, BPW)], uidx_v)
    pltpu.sync_copy(iidx_hbm.at[pl.ds(base, BPW)], iidx_v)

    for s in range(NSLOT):
        _issue_pair(euT, eiT, uslots.at[s], islots.at[s], sems[s],
                    uidx_v, iidx_v, s)

    lane = lax.iota(jnp.int32, L)

    def make_branch(s):
        def branch(r, out_vec):
            uslot = uslots.at[s]
            islot = islots.at[s]
            pltpu.make_async_copy(euT.at[:, pl.ds(0, 128)], uslot, sems[s]).wait()
            pltpu.make_async_copy(eiT.at[:, pl.ds(0, 128)], islot, sems[s]).wait()
            u = _scalar_at(uidx_v, r)
            i = _scalar_at(iidx_v, r)
            ul = jnp.full((L,), u % 128, jnp.int32)
            il = jnp.full((L,), i % 128, jnp.int32)
            acc = jnp.zeros((L,), jnp.float32)
            for a in range(F // L):
                rows = lane + (a * L)
                vu = plsc.load_gather(uslot, [rows, ul])
                vi = plsc.load_gather(islot, [rows, il])
                acc = acc + vu * vi
            sval = jnp.sum(acc)
            out_vec = jnp.where(lane == (r % L), sval, out_vec)
            e = r + NSLOT

            @pl.when(e < BPW)
            def _():
                _issue_pair(euT, eiT, uslot, islot, sems[s], uidx_v, iidx_v, e)
            return out_vec
        return branch

    branches = [make_branch(s) for s in range(NSLOT)]

    def body(r, out_vec):
        out_vec = lax.switch(r % NSLOT, branches, r, out_vec)

        @pl.when(r % L == L - 1)
        def _():
            g16 = pl.multiple_of((r // L) * L, L)
            out_v[pl.ds(g16, L)] = out_vec
        return jnp.where(r % L == L - 1, jnp.zeros((L,), jnp.float32), out_vec)

    lax.fori_loop(0, BPW, body, jnp.zeros((L,), jnp.float32))
    pltpu.sync_copy(out_v, out_hbm.at[pl.ds(base, BPW)])


@functools.partial(
    pl.kernel,
    mesh=plsc.VectorSubcoreMesh(core_axis_name="c", subcore_axis_name="s"),
    out_type=jax.ShapeDtypeStruct((B,), jnp.float32),
    compiler_params=pltpu.CompilerParams(
        needs_layout_passes=False, use_tc_tiling_on_sc=True),
    scratch_types=[
        pltpu.VMEM((BPW,), jnp.int32),
        pltpu.VMEM((BPW,), jnp.int32),
        pltpu.VMEM((NSLOT, F, 128), jnp.float32),
        pltpu.VMEM((NSLOT, F, 128), jnp.float32),
        pltpu.VMEM((BPW,), jnp.float32),
        [pltpu.SemaphoreType.DMA] * NSLOT,
    ],
)
def _mf_dots(uidx_hbm, iidx_hbm, euT, eiT, out_hbm,
             uidx_v, iidx_v, uslots, islots, out_v, sems):
    _dots_body(uidx_hbm, iidx_hbm, euT, eiT, out_hbm,
               uidx_v, iidx_v, uslots, islots, out_v, sems)


def _bias_body(dots_hbm, uidx_hbm, iidx_hbm, bu_hbm, bi_hbm, out_hbm,
               uidx_v, iidx_v, dots_v, bu_v, bi_v, out_v, sem):
    wid = lax.axis_index("s") * NC + lax.axis_index("c")
    base = wid * BPW

    pltpu.sync_copy(uidx_hbm.at[pl.ds(base, BPW)], uidx_v)
    pltpu.sync_copy(iidx_hbm.at[pl.ds(base, BPW)], iidx_v)
    pltpu.sync_copy(dots_hbm.at[pl.ds(base, BPW)], dots_v)

    copies = []
    for c in range(NCHUNK):
        sl = pl.ds(c * CHUNK, CHUNK)
        copies.append(pltpu.async_copy(bu_hbm.at[uidx_v.at[sl]], bu_v.at[sl], sem))
        copies.append(pltpu.async_copy(bi_hbm.at[iidx_v.at[sl]], bi_v.at[sl], sem))
    for cp in copies:
        cp.wait()

    def grp(g, _):
        sl = pl.ds(g * L, L)
        out_v[sl] = dots_v[sl] + bu_v[sl] + bi_v[sl] + jnp.float32(MU)
        return 0
    lax.fori_loop(0, BPW // L, grp, 0)
    pltpu.sync_copy(out_v, out_hbm.at[pl.ds(base, BPW)])


@functools.partial(
    pl.kernel,
    mesh=plsc.VectorSubcoreMesh(core_axis_name="c", subcore_axis_name="s"),
    out_type=jax.ShapeDtypeStruct((B,), jnp.float32),
    compiler_params=pltpu.CompilerParams(
        needs_layout_passes=False, use_tc_tiling_on_sc=True),
    scratch_types=[
        pltpu.VMEM((BPW,), jnp.int32),
        pltpu.VMEM((BPW,), jnp.int32),
        pltpu.VMEM((BPW,), jnp.float32),
        pltpu.VMEM((BPW,), jnp.float32),
        pltpu.VMEM((BPW,), jnp.float32),
        pltpu.VMEM((BPW,), jnp.float32),
        pltpu.SemaphoreType.DMA,
    ],
)
def _mf_bias(dots_hbm, uidx_hbm, iidx_hbm, bu_hbm, bi_hbm, out_hbm,
             uidx_v, iidx_v, dots_v, bu_v, bi_v, out_v, sem):
    _bias_body(dots_hbm, uidx_hbm, iidx_hbm, bu_hbm, bi_hbm, out_hbm,
               uidx_v, iidx_v, dots_v, bu_v, bi_v, out_v, sem)


def kernel(user_idx, item_idx, embed_user, embed_item, user_bias, item_bias):
    dots = _mf_dots(user_idx, item_idx, embed_user.T, embed_item.T)
    return _mf_bias(dots, user_idx, item_idx, user_bias[:, 0], item_bias[:, 0])
